# baseline (speedup 1.0000x reference)
import jax
import jax.numpy as jnp
from jax import lax
from jax.experimental import pallas as pl
from jax.experimental.pallas import tpu as pltpu
from jax.experimental.pallas import tpu_sc as plsc

B = 16384
DIM = 64
PDIM = 128
NUM_CORES = 2
NUM_SUBCORES = 16
NW = NUM_CORES * NUM_SUBCORES
BPW = B // NW
CHUNK = 256
ROWS_PER_BLOCK = 2048


def _gather_body(rt_ref, ln_ref, tp_ref, wr_ref, wl_ref, wt_ref,
                 o0_ref, o1_ref, o2_ref,
                 idx0, idx1, idx2, rows0, rows1, rows2, sem):
    wid = lax.axis_index("s") * NUM_CORES + lax.axis_index("c")

    for half in range(BPW // CHUNK):
        base = wid * BPW + half * CHUNK
        pltpu.sync_copy(rt_ref.at[pl.ds(base, CHUNK)], idx0)
        pltpu.sync_copy(ln_ref.at[pl.ds(base, CHUNK)], idx1)
        pltpu.sync_copy(tp_ref.at[pl.ds(base, CHUNK)], idx2)

        c0 = pltpu.async_copy(wr_ref.at[idx0], rows0, sem)
        c1 = pltpu.async_copy(wl_ref.at[idx1], rows1, sem)
        c2 = pltpu.async_copy(wt_ref.at[idx2], rows2, sem)
        c0.wait()
        pltpu.sync_copy(rows0, o0_ref.at[pl.ds(base, CHUNK)])
        c1.wait()
        pltpu.sync_copy(rows1, o1_ref.at[pl.ds(base, CHUNK)])
        c2.wait()
        pltpu.sync_copy(rows2, o2_ref.at[pl.ds(base, CHUNK)])


def _concat_body(a_ref, b_ref, c_ref, out_ref):
    out_ref[...] = jnp.concatenate(
        [a_ref[:, :DIM], b_ref[:, :DIM], c_ref[:, :DIM]], axis=-1)


@jax.jit
def _lookup_concat(road_type, lane, time_period, W_road, W_lane, W_time):
    pad = [(0, 0), (0, PDIM - DIM)]
    wr = jnp.pad(W_road, pad)
    wl = jnp.pad(W_lane, pad)
    wt = jnp.pad(W_time, pad)

    mesh = plsc.VectorSubcoreMesh(core_axis_name="c", subcore_axis_name="s")
    emb_shape = jax.ShapeDtypeStruct((B, PDIM), jnp.float32)
    e0, e1, e2 = pl.kernel(
        _gather_body,
        out_type=(emb_shape, emb_shape, emb_shape),
        mesh=mesh,
        scratch_types=[
            pltpu.VMEM((CHUNK,), jnp.int32),
            pltpu.VMEM((CHUNK,), jnp.int32),
            pltpu.VMEM((CHUNK,), jnp.int32),
            pltpu.VMEM((CHUNK, PDIM), jnp.float32),
            pltpu.VMEM((CHUNK, PDIM), jnp.float32),
            pltpu.VMEM((CHUNK, PDIM), jnp.float32),
            pltpu.SemaphoreType.DMA,
        ],
    )(road_type, lane, time_period, wr, wl, wt)

    block = ROWS_PER_BLOCK
    return pl.pallas_call(
        _concat_body,
        grid=(B // block,),
        in_specs=[
            pl.BlockSpec((block, PDIM), lambda i: (i, 0)),
            pl.BlockSpec((block, PDIM), lambda i: (i, 0)),
            pl.BlockSpec((block, PDIM), lambda i: (i, 0)),
        ],
        out_specs=pl.BlockSpec((block, 3 * DIM), lambda i: (i, 0)),
        out_shape=jax.ShapeDtypeStruct((B, 3 * DIM), jnp.float32),
    )(e0, e1, e2)


def kernel(road_type, lane, time_period, W_road, W_lane, W_time):
    return _lookup_concat(
        road_type.astype(jnp.int32),
        lane.astype(jnp.int32),
        time_period.astype(jnp.int32),
        W_road, W_lane, W_time,
    )


# speedup vs baseline: 1.0262x; 1.0262x over previous
import jax
import jax.numpy as jnp
from jax import lax
from jax.experimental import pallas as pl
from jax.experimental.pallas import tpu as pltpu
from jax.experimental.pallas import tpu_sc as plsc

B = 16384
DIM = 64
PDIM = 128
NUM_CORES = 2
NUM_SUBCORES = 16
NW = NUM_CORES * NUM_SUBCORES
BPW = B // NW
CHUNK = 256
ROWS_PER_BLOCK = 2048


def _gather_body(rt_ref, ln_ref, tp_ref, wr_ref, wl_ref, wt_ref,
                 o0_ref, o1_ref, o2_ref,
                 idx0, idx1, idx2, rows0, rows1, rows2, sem):
    wid = lax.axis_index("s") * NUM_CORES + lax.axis_index("c")

    for half in range(BPW // CHUNK):
        base = wid * BPW + half * CHUNK
        pltpu.sync_copy(rt_ref.at[pl.ds(base, CHUNK)], idx0)
        pltpu.sync_copy(ln_ref.at[pl.ds(base, CHUNK)], idx1)
        pltpu.sync_copy(tp_ref.at[pl.ds(base, CHUNK)], idx2)

        c0 = pltpu.async_copy(wr_ref.at[idx0], rows0, sem)
        c1 = pltpu.async_copy(wl_ref.at[idx1], rows1, sem)
        c2 = pltpu.async_copy(wt_ref.at[idx2], rows2, sem)
        c0.wait()
        pltpu.sync_copy(rows0, o0_ref.at[pl.ds(base, CHUNK)])
        c1.wait()
        pltpu.sync_copy(rows1, o1_ref.at[pl.ds(base, CHUNK)])
        c2.wait()
        pltpu.sync_copy(rows2, o2_ref.at[pl.ds(base, CHUNK)])


def _concat_body(a_ref, b_ref, c_ref, out_ref):
    out_ref[...] = jnp.concatenate(
        [a_ref[:, :DIM], b_ref[:, :DIM], c_ref[:, :DIM]], axis=-1)


@jax.jit
def _lookup_concat(road_type, lane, time_period, W_road, W_lane, W_time):
    pad = [(0, 0), (0, PDIM - DIM)]
    wr = jnp.pad(W_road, pad)
    wl = jnp.pad(W_lane, pad)
    wt = jnp.pad(W_time, pad)

    mesh = plsc.VectorSubcoreMesh(core_axis_name="c", subcore_axis_name="s")
    emb_shape = jax.ShapeDtypeStruct((B, PDIM), jnp.float32)
    e0, e1, e2 = pl.kernel(
        _gather_body,
        out_type=(emb_shape, emb_shape, emb_shape),
        mesh=mesh,
        scratch_types=[
            pltpu.VMEM((CHUNK,), jnp.int32),
            pltpu.VMEM((CHUNK,), jnp.int32),
            pltpu.VMEM((CHUNK,), jnp.int32),
            pltpu.VMEM((CHUNK, PDIM), jnp.float32),
            pltpu.VMEM((CHUNK, PDIM), jnp.float32),
            pltpu.VMEM((CHUNK, PDIM), jnp.float32),
            pltpu.SemaphoreType.DMA,
        ],
    )(road_type, lane, time_period, wr, wl, wt)

    return jnp.concatenate(
        [e0[:, :DIM], e1[:, :DIM], e2[:, :DIM]], axis=-1)


def kernel(road_type, lane, time_period, W_road, W_lane, W_time):
    return _lookup_concat(
        road_type.astype(jnp.int32),
        lane.astype(jnp.int32),
        time_period.astype(jnp.int32),
        W_road, W_lane, W_time,
    )
